# Initial kernel scaffold; baseline (speedup 1.0000x reference)
import jax
import jax.numpy as jnp
from jax.experimental import pallas as pl


def kernel(x, conv_w, conv_b, sn_u, bn_gamma, bn_beta):
    raise NotImplementedError("write your pallas kernel here")


# speedup vs baseline: 1.0653x; 1.0653x over previous
import functools

import jax
import jax.numpy as jnp
from jax.experimental import pallas as pl
from jax.experimental.pallas import tpu as pltpu

_BN_EPS = 1e-5
_SN_EPS = 1e-12


def _conv_taps(xp_ref, w_ref, *, H, W, Cin, C2):
    accs = []
    for pi in range(2):
        acc = None
        for a2 in range(2):
            for bb in range(3):
                t = (pi * 2 + a2) * 3 + bb
                tap = xp_ref[0, pl.ds(pi + a2, H), pl.ds(bb, W), :]
                d = jnp.dot(tap.reshape(H * W, Cin), w_ref[t],
                            preferred_element_type=jnp.float32)
                acc = d if acc is None else acc + d
        accs.append(acc)
    return accs


def _stats_kernel(x_ref, w_ref, b_ref, xp_ref, st_ref, *, H, W, Cin, C2):
    xp_ref[...] = jnp.zeros_like(xp_ref)
    xp_ref[0, pl.ds(1, H), pl.ds(1, W), :] = x_ref[0].astype(jnp.bfloat16)
    acc0, acc1 = _conv_taps(xp_ref, w_ref, H=H, W=W, Cin=Cin, C2=C2)
    acc0 = acc0 + b_ref[...]
    acc1 = acc1 + b_ref[...]
    ssum = (jnp.sum(acc0, axis=0, keepdims=True)
            + jnp.sum(acc1, axis=0, keepdims=True))
    ssq = (jnp.sum(acc0 * acc0, axis=0, keepdims=True)
           + jnp.sum(acc1 * acc1, axis=0, keepdims=True))
    st_ref[...] = jnp.concatenate([ssum, ssq], axis=0).reshape(1, 1, 2, C2)


def _apply_kernel(xp_ref, w_ref, sc_ref, sh_ref, o_ref, *, H, W, Cin, C2):
    acc0, acc1 = _conv_taps(xp_ref, w_ref, H=H, W=W, Cin=Cin, C2=C2)
    sc = sc_ref[...]
    sh = sh_ref[...]
    o0 = jnp.maximum(acc0 * sc + sh, 0.0).reshape(H, W, C2)
    o1 = jnp.maximum(acc1 * sc + sh, 0.0).reshape(H, W, C2)
    o_ref[...] = jnp.stack([o0, o1], axis=1)[None]


def _spectral_norm_weight(w, u):
    Cout = w.shape[-1]
    w_mat = w.reshape(-1, Cout).T
    v = w_mat.T @ u
    v = v / (jnp.linalg.norm(v) + _SN_EPS)
    u_new = w_mat @ v
    u_new = u_new / (jnp.linalg.norm(u_new) + _SN_EPS)
    sigma = jnp.dot(u_new, w_mat @ v)
    return w / sigma


def _fold_upsample_weights(w):
    z = jnp.zeros_like(w[:, 0])
    wc0 = jnp.stack([w[:, 0], w[:, 1] + w[:, 2], z], axis=1)
    wc1 = jnp.stack([z, w[:, 0] + w[:, 1], w[:, 2]], axis=1)
    wc = jnp.concatenate([wc0, wc1], axis=-1)
    w0 = jnp.stack([wc[0], wc[1] + wc[2]], axis=0)
    w1 = jnp.stack([wc[0] + wc[1], wc[2]], axis=0)
    w_eff = jnp.stack([w0, w1], axis=0)
    Cin, C2 = w.shape[2], 2 * w.shape[3]
    return w_eff.reshape(12, Cin, C2).astype(jnp.bfloat16)


def kernel(x, conv_w, conv_b, sn_u, bn_gamma, bn_beta):
    N, H, W, Cin = x.shape
    Cout = conv_w.shape[-1]
    H2, W2 = 2 * H, 2 * W
    C2 = 2 * Cout
    Hp, Wp = H + 2, W + 2

    w_sn = _spectral_norm_weight(conv_w, sn_u)
    w_eff = _fold_upsample_weights(w_sn)
    bias2 = jnp.tile(conv_b, 2).reshape(1, C2).astype(jnp.float32)

    vmem_limit = 100 * 1024 * 1024

    xp, stats = pl.pallas_call(
        functools.partial(_stats_kernel, H=H, W=W, Cin=Cin, C2=C2),
        out_shape=(
            jax.ShapeDtypeStruct((N, Hp, Wp, Cin), jnp.bfloat16),
            jax.ShapeDtypeStruct((N, 1, 2, C2), jnp.float32),
        ),
        grid=(N,),
        in_specs=[
            pl.BlockSpec((1, H, W, Cin), lambda n: (n, 0, 0, 0)),
            pl.BlockSpec((12, Cin, C2), lambda n: (0, 0, 0)),
            pl.BlockSpec((1, C2), lambda n: (0, 0)),
        ],
        out_specs=(
            pl.BlockSpec((1, Hp, Wp, Cin), lambda n: (n, 0, 0, 0)),
            pl.BlockSpec((1, 1, 2, C2), lambda n: (n, 0, 0, 0)),
        ),
        compiler_params=pltpu.CompilerParams(
            dimension_semantics=("parallel",),
            vmem_limit_bytes=vmem_limit),
    )(x, w_eff, bias2)

    tot = jnp.sum(stats, axis=(0, 1)).reshape(2, 2, Cout).sum(axis=1)
    cnt = float(N * H2 * W2)
    mean = tot[0] / cnt
    var = jnp.maximum(tot[1] / cnt - mean * mean, 0.0)
    invstd = jax.lax.rsqrt(var + _BN_EPS)
    scale = bn_gamma * invstd
    shift = bn_beta - mean * scale
    scale2 = jnp.tile(scale, 2).reshape(1, C2).astype(jnp.float32)
    shift2 = (jnp.tile(shift, 2).reshape(1, C2) + bias2 * scale2
              ).astype(jnp.float32)

    out = pl.pallas_call(
        functools.partial(_apply_kernel, H=H, W=W, Cin=Cin, C2=C2),
        out_shape=jax.ShapeDtypeStruct((N, H, 2, W, C2), jnp.float32),
        grid=(N,),
        in_specs=[
            pl.BlockSpec((1, Hp, Wp, Cin), lambda n: (n, 0, 0, 0)),
            pl.BlockSpec((12, Cin, C2), lambda n: (0, 0, 0)),
            pl.BlockSpec((1, C2), lambda n: (0, 0)),
            pl.BlockSpec((1, C2), lambda n: (0, 0)),
        ],
        out_specs=pl.BlockSpec((1, H, 2, W, C2), lambda n: (n, 0, 0, 0, 0)),
        compiler_params=pltpu.CompilerParams(
            dimension_semantics=("parallel",),
            vmem_limit_bytes=vmem_limit),
    )(xp, w_eff, scale2, shift2)

    return out.reshape(N, H2, W2, Cout)


# speedup vs baseline: 1.3508x; 1.2681x over previous
import functools

import jax
import jax.numpy as jnp
from jax.experimental import pallas as pl
from jax.experimental.pallas import tpu as pltpu

_BN_EPS = 1e-5
_SN_EPS = 1e-12


def _conv_accs(x2d, w_ref, *, H, W, Cin):
    HW = H * W
    z = jnp.zeros((W, Cin), jnp.float32)
    xe = jnp.concatenate([z, x2d, z], axis=0)
    xc = jnp.concatenate([xe[0:HW + W], xe[W:HW + 2 * W]], axis=1)
    col = jax.lax.broadcasted_iota(jnp.int32, (HW + W, 2 * Cin), 0)
    col = (col & (W - 1)) if (W & (W - 1)) == 0 else (col % W)
    p1 = xc.astype(jnp.bfloat16)
    p0 = jnp.where(col != 0, pltpu.roll(xc, 1, axis=0), 0.0
                   ).astype(jnp.bfloat16)
    p2 = jnp.where(col != W - 1, pltpu.roll(xc, HW + W - 1, axis=0), 0.0
                   ).astype(jnp.bfloat16)
    accs = []
    for pi in range(2):
        acc = None
        for bb, p in ((0, p0), (1, p1), (2, p2)):
            d = jnp.dot(p[pi * W:pi * W + HW], w_ref[pi * 3 + bb],
                        preferred_element_type=jnp.float32)
            acc = d if acc is None else acc + d
        accs.append(acc)
    return accs


def _stats_kernel(x_ref, w_ref, b_ref, st_ref, *, H, W, Cin, C2):
    x2d = x_ref[0].reshape(H * W, Cin)
    acc0, acc1 = _conv_accs(x2d, w_ref, H=H, W=W, Cin=Cin)
    acc0 = acc0 + b_ref[...]
    acc1 = acc1 + b_ref[...]
    ssum = (jnp.sum(acc0, axis=0, keepdims=True)
            + jnp.sum(acc1, axis=0, keepdims=True))
    ssq = (jnp.sum(acc0 * acc0, axis=0, keepdims=True)
           + jnp.sum(acc1 * acc1, axis=0, keepdims=True))
    st_ref[...] = jnp.concatenate([ssum, ssq], axis=0).reshape(1, 1, 2, C2)


def _apply_kernel(x_ref, w_ref, sc_ref, sh_ref, o_ref, *, H, W, Cin, C2):
    x2d = x_ref[0].reshape(H * W, Cin)
    acc0, acc1 = _conv_accs(x2d, w_ref, H=H, W=W, Cin=Cin)
    sc = sc_ref[...]
    sh = sh_ref[...]
    o0 = jnp.maximum(acc0 * sc + sh, 0.0).reshape(H, W, C2)
    o1 = jnp.maximum(acc1 * sc + sh, 0.0).reshape(H, W, C2)
    o_ref[...] = jnp.stack([o0, o1], axis=1)[None]


def _spectral_norm_weight(w, u):
    Cout = w.shape[-1]
    w_mat = w.reshape(-1, Cout).T
    v = w_mat.T @ u
    v = v / (jnp.linalg.norm(v) + _SN_EPS)
    u_new = w_mat @ v
    u_new = u_new / (jnp.linalg.norm(u_new) + _SN_EPS)
    sigma = jnp.dot(u_new, w_mat @ v)
    return w / sigma


def _fold_upsample_weights(w):
    z = jnp.zeros_like(w[:, 0])
    wc0 = jnp.stack([w[:, 0], w[:, 1] + w[:, 2], z], axis=1)
    wc1 = jnp.stack([z, w[:, 0] + w[:, 1], w[:, 2]], axis=1)
    wc = jnp.concatenate([wc0, wc1], axis=-1)
    w0 = jnp.stack([wc[0], wc[1] + wc[2]], axis=1)
    w1 = jnp.stack([wc[0] + wc[1], wc[2]], axis=1)
    w2 = jnp.stack([w0, w1], axis=0)
    Cin, C2 = w.shape[2], 2 * w.shape[3]
    return w2.reshape(6, 2 * Cin, C2).astype(jnp.bfloat16)


def kernel(x, conv_w, conv_b, sn_u, bn_gamma, bn_beta):
    N, H, W, Cin = x.shape
    Cout = conv_w.shape[-1]
    H2, W2 = 2 * H, 2 * W
    C2 = 2 * Cout

    w_sn = _spectral_norm_weight(conv_w, sn_u)
    w2 = _fold_upsample_weights(w_sn)
    bias2 = jnp.tile(conv_b, 2).reshape(1, C2).astype(jnp.float32)

    vmem_limit = 100 * 1024 * 1024

    stats = pl.pallas_call(
        functools.partial(_stats_kernel, H=H, W=W, Cin=Cin, C2=C2),
        out_shape=jax.ShapeDtypeStruct((N, 1, 2, C2), jnp.float32),
        grid=(N,),
        in_specs=[
            pl.BlockSpec((1, H, W, Cin), lambda n: (n, 0, 0, 0)),
            pl.BlockSpec((6, 2 * Cin, C2), lambda n: (0, 0, 0)),
            pl.BlockSpec((1, C2), lambda n: (0, 0)),
        ],
        out_specs=pl.BlockSpec((1, 1, 2, C2), lambda n: (n, 0, 0, 0)),
        compiler_params=pltpu.CompilerParams(
            dimension_semantics=("parallel",),
            vmem_limit_bytes=vmem_limit),
    )(x, w2, bias2)

    tot = jnp.sum(stats, axis=(0, 1)).reshape(2, 2, Cout).sum(axis=1)
    cnt = float(N * H2 * W2)
    mean = tot[0] / cnt
    var = jnp.maximum(tot[1] / cnt - mean * mean, 0.0)
    invstd = jax.lax.rsqrt(var + _BN_EPS)
    scale = bn_gamma * invstd
    shift = bn_beta - mean * scale
    scale2 = jnp.tile(scale, 2).reshape(1, C2).astype(jnp.float32)
    shift2 = (jnp.tile(shift, 2).reshape(1, C2) + bias2 * scale2
              ).astype(jnp.float32)

    out = pl.pallas_call(
        functools.partial(_apply_kernel, H=H, W=W, Cin=Cin, C2=C2),
        out_shape=jax.ShapeDtypeStruct((N, H, 2, W, C2), jnp.float32),
        grid=(N,),
        in_specs=[
            pl.BlockSpec((1, H, W, Cin), lambda n: (n, 0, 0, 0)),
            pl.BlockSpec((6, 2 * Cin, C2), lambda n: (0, 0, 0)),
            pl.BlockSpec((1, C2), lambda n: (0, 0)),
            pl.BlockSpec((1, C2), lambda n: (0, 0)),
        ],
        out_specs=pl.BlockSpec((1, H, 2, W, C2), lambda n: (n, 0, 0, 0, 0)),
        compiler_params=pltpu.CompilerParams(
            dimension_semantics=("parallel",),
            vmem_limit_bytes=vmem_limit),
    )(x, w2, scale2, shift2)

    return out.reshape(N, H2, W2, Cout)


# speedup vs baseline: 1.3589x; 1.0060x over previous
import functools

import jax
import jax.numpy as jnp
from jax.experimental import pallas as pl
from jax.experimental.pallas import tpu as pltpu

_BN_EPS = 1e-5
_SN_EPS = 1e-12


def _conv_accs(x2d, w_ref, *, H, W, Cin):
    HW = H * W
    z = jnp.zeros((W, Cin), jnp.float32)
    xe = jnp.concatenate([z, x2d, z], axis=0)
    xc = jnp.concatenate([xe[0:HW + W], xe[W:HW + 2 * W]], axis=1)
    col = jax.lax.broadcasted_iota(jnp.int32, (HW + W, 2 * Cin), 0)
    col = (col & (W - 1)) if (W & (W - 1)) == 0 else (col % W)
    p1 = xc.astype(jnp.bfloat16)
    p0 = jnp.where(col != 0, pltpu.roll(xc, 1, axis=0), 0.0
                   ).astype(jnp.bfloat16)
    p2 = jnp.where(col != W - 1, pltpu.roll(xc, HW + W - 1, axis=0), 0.0
                   ).astype(jnp.bfloat16)
    accs = []
    for pi in range(2):
        acc = None
        for bb, p in ((0, p0), (1, p1), (2, p2)):
            d = jnp.dot(p[pi * W:pi * W + HW], w_ref[pi * 3 + bb],
                        preferred_element_type=jnp.float32)
            acc = d if acc is None else acc + d
        accs.append(acc)
    return accs


def _stats_kernel(x_ref, w_ref, b_ref, st_ref, *, H, W, Cin, C2):
    n = pl.program_id(0)
    x2d = x_ref[0].reshape(H * W, Cin)
    acc0, acc1 = _conv_accs(x2d, w_ref, H=H, W=W, Cin=Cin)
    acc0 = acc0 + b_ref[...]
    acc1 = acc1 + b_ref[...]
    ssum = (jnp.sum(acc0, axis=0, keepdims=True)
            + jnp.sum(acc1, axis=0, keepdims=True))
    ssq = (jnp.sum(acc0 * acc0, axis=0, keepdims=True)
           + jnp.sum(acc1 * acc1, axis=0, keepdims=True))
    part = jnp.concatenate([ssum, ssq], axis=0)

    @pl.when(n == 0)
    def _():
        st_ref[...] = jnp.zeros_like(st_ref)

    st_ref[...] += part


def _apply_kernel(x_ref, w_ref, st_ref, gbb_ref, o_ref, *, H, W, Cin, C2,
                  cnt):
    Cout = C2 // 2
    st = st_ref[...]
    tot = st[:, 0:Cout] + st[:, Cout:C2]
    mean = tot[0:1] * (1.0 / cnt)
    var = jnp.maximum(tot[1:2] * (1.0 / cnt) - mean * mean, 0.0)
    invstd = jax.lax.rsqrt(var + _BN_EPS)
    gamma, beta, bias = gbb_ref[0:1], gbb_ref[1:2], gbb_ref[2:3]
    scale = gamma * invstd
    shift = beta + (bias - mean) * scale
    sc = jnp.concatenate([scale, scale], axis=1)
    sh = jnp.concatenate([shift, shift], axis=1)

    x2d = x_ref[0].reshape(H * W, Cin)
    acc0, acc1 = _conv_accs(x2d, w_ref, H=H, W=W, Cin=Cin)
    o0 = jnp.maximum(acc0 * sc + sh, 0.0).reshape(H, W, C2)
    o1 = jnp.maximum(acc1 * sc + sh, 0.0).reshape(H, W, C2)
    o_ref[...] = jnp.stack([o0, o1], axis=1)[None]


def _spectral_norm_weight(w, u):
    Cout = w.shape[-1]
    w_mat = w.reshape(-1, Cout).T
    v = w_mat.T @ u
    v = v / (jnp.linalg.norm(v) + _SN_EPS)
    u_new = w_mat @ v
    u_new = u_new / (jnp.linalg.norm(u_new) + _SN_EPS)
    sigma = jnp.dot(u_new, w_mat @ v)
    return w / sigma


def _fold_upsample_weights(w):
    z = jnp.zeros_like(w[:, 0])
    wc0 = jnp.stack([w[:, 0], w[:, 1] + w[:, 2], z], axis=1)
    wc1 = jnp.stack([z, w[:, 0] + w[:, 1], w[:, 2]], axis=1)
    wc = jnp.concatenate([wc0, wc1], axis=-1)
    w0 = jnp.stack([wc[0], wc[1] + wc[2]], axis=1)
    w1 = jnp.stack([wc[0] + wc[1], wc[2]], axis=1)
    w2 = jnp.stack([w0, w1], axis=0)
    Cin, C2 = w.shape[2], 2 * w.shape[3]
    return w2.reshape(6, 2 * Cin, C2).astype(jnp.bfloat16)


def kernel(x, conv_w, conv_b, sn_u, bn_gamma, bn_beta):
    N, H, W, Cin = x.shape
    Cout = conv_w.shape[-1]
    H2, W2 = 2 * H, 2 * W
    C2 = 2 * Cout

    w_sn = _spectral_norm_weight(conv_w, sn_u)
    w2 = _fold_upsample_weights(w_sn)
    bias2 = jnp.tile(conv_b, 2).reshape(1, C2).astype(jnp.float32)
    gbb = jnp.stack([bn_gamma, bn_beta, conv_b]).astype(jnp.float32)

    vmem_limit = 100 * 1024 * 1024

    stats = pl.pallas_call(
        functools.partial(_stats_kernel, H=H, W=W, Cin=Cin, C2=C2),
        out_shape=jax.ShapeDtypeStruct((2, C2), jnp.float32),
        grid=(N,),
        in_specs=[
            pl.BlockSpec((1, H, W, Cin), lambda n: (n, 0, 0, 0)),
            pl.BlockSpec((6, 2 * Cin, C2), lambda n: (0, 0, 0)),
            pl.BlockSpec((1, C2), lambda n: (0, 0)),
        ],
        out_specs=pl.BlockSpec((2, C2), lambda n: (0, 0)),
        compiler_params=pltpu.CompilerParams(
            dimension_semantics=("arbitrary",),
            vmem_limit_bytes=vmem_limit),
    )(x, w2, bias2)

    out = pl.pallas_call(
        functools.partial(_apply_kernel, H=H, W=W, Cin=Cin, C2=C2,
                          cnt=float(N * H2 * W2)),
        out_shape=jax.ShapeDtypeStruct((N, H, 2, W, C2), jnp.float32),
        grid=(N,),
        in_specs=[
            pl.BlockSpec((1, H, W, Cin), lambda n: (n, 0, 0, 0)),
            pl.BlockSpec((6, 2 * Cin, C2), lambda n: (0, 0, 0)),
            pl.BlockSpec((2, C2), lambda n: (0, 0)),
            pl.BlockSpec((3, Cout), lambda n: (0, 0)),
        ],
        out_specs=pl.BlockSpec((1, H, 2, W, C2), lambda n: (n, 0, 0, 0, 0)),
        compiler_params=pltpu.CompilerParams(
            dimension_semantics=("arbitrary",),
            vmem_limit_bytes=vmem_limit),
    )(x, w2, stats, gbb)

    return out.reshape(N, H2, W2, Cout)


# speedup vs baseline: 1.3847x; 1.0190x over previous
import functools

import jax
import jax.numpy as jnp
from jax.experimental import pallas as pl
from jax.experimental.pallas import tpu as pltpu

_BN_EPS = 1e-5
_SN_EPS = 1e-12


def _fold_weights(w, u_row):
    Cin, Cout = w.shape[2], w.shape[3]
    wf = w.reshape(9 * Cin, Cout)
    v = jnp.sum(wf * u_row, axis=1, keepdims=True)
    v = v / (jnp.sqrt(jnp.sum(v * v)) + _SN_EPS)
    wv = jnp.sum(wf * v, axis=0, keepdims=True)
    u_new = wv / (jnp.sqrt(jnp.sum(wv * wv)) + _SN_EPS)
    sigma = jnp.sum(u_new * wv)
    w = w / sigma
    d0, d1, d2 = w[:, 0], w[:, 1], w[:, 2]
    z = jnp.zeros_like(d0)
    e = (jnp.concatenate([d0, z], axis=2),
         jnp.concatenate([d1 + d2, d0 + d1], axis=2),
         jnp.concatenate([z, d2], axis=2))
    pieces = []
    for pi in range(2):
        for bb in range(3):
            if pi == 0:
                t0, t1 = e[bb][0], e[bb][1] + e[bb][2]
            else:
                t0, t1 = e[bb][0] + e[bb][1], e[bb][2]
            pieces.append(jnp.concatenate([t0, t1], axis=0))
    return jnp.stack(pieces, axis=0).astype(jnp.bfloat16)


def _conv_accs(x2d, w_ref, *, H, W, Cin):
    HW = H * W
    z = jnp.zeros((W, Cin), jnp.float32)
    xe = jnp.concatenate([z, x2d, z], axis=0)
    xc = jnp.concatenate([xe[0:HW + W], xe[W:HW + 2 * W]], axis=1)
    col = jax.lax.broadcasted_iota(jnp.int32, (HW + W, 2 * Cin), 0)
    col = (col & (W - 1)) if (W & (W - 1)) == 0 else (col % W)
    p1 = xc.astype(jnp.bfloat16)
    p0 = jnp.where(col != 0, pltpu.roll(xc, 1, axis=0), 0.0
                   ).astype(jnp.bfloat16)
    p2 = jnp.where(col != W - 1, pltpu.roll(xc, HW + W - 1, axis=0), 0.0
                   ).astype(jnp.bfloat16)
    accs = []
    for pi in range(2):
        acc = None
        for bb, p in ((0, p0), (1, p1), (2, p2)):
            d = jnp.dot(p[pi * W:pi * W + HW], w_ref[pi * 3 + bb],
                        preferred_element_type=jnp.float32)
            acc = d if acc is None else acc + d
        accs.append(acc)
    return accs


def _stats_kernel(x_ref, cw_ref, u_ref, b_ref, st_ref, w2_ref,
                  *, H, W, Cin, C2):
    n = pl.program_id(0)

    @pl.when(n == 0)
    def _():
        w2_ref[...] = _fold_weights(cw_ref[...], u_ref[...])
        st_ref[...] = jnp.zeros_like(st_ref)

    x2d = x_ref[0].reshape(H * W, Cin)
    acc0, acc1 = _conv_accs(x2d, w2_ref, H=H, W=W, Cin=Cin)
    b2 = jnp.concatenate([b_ref[...], b_ref[...]], axis=1)
    acc0 = acc0 + b2
    acc1 = acc1 + b2
    ssum = (jnp.sum(acc0, axis=0, keepdims=True)
            + jnp.sum(acc1, axis=0, keepdims=True))
    ssq = (jnp.sum(acc0 * acc0, axis=0, keepdims=True)
           + jnp.sum(acc1 * acc1, axis=0, keepdims=True))
    st_ref[...] += jnp.concatenate([ssum, ssq], axis=0)


def _apply_kernel(x_ref, w_ref, st_ref, g_ref, bt_ref, b_ref, o_ref,
                  *, H, W, Cin, C2, cnt):
    Cout = C2 // 2
    st = st_ref[...]
    tot = st[:, 0:Cout] + st[:, Cout:C2]
    mean = tot[0:1] * (1.0 / cnt)
    var = jnp.maximum(tot[1:2] * (1.0 / cnt) - mean * mean, 0.0)
    invstd = jax.lax.rsqrt(var + _BN_EPS)
    scale = g_ref[...] * invstd
    shift = bt_ref[...] + (b_ref[...] - mean) * scale
    sc = jnp.concatenate([scale, scale], axis=1)
    sh = jnp.concatenate([shift, shift], axis=1)

    x2d = x_ref[0].reshape(H * W, Cin)
    acc0, acc1 = _conv_accs(x2d, w_ref, H=H, W=W, Cin=Cin)
    o0 = jnp.maximum(acc0 * sc + sh, 0.0).reshape(H, W, C2)
    o1 = jnp.maximum(acc1 * sc + sh, 0.0).reshape(H, W, C2)
    o_ref[...] = jnp.stack([o0, o1], axis=1)[None]


def kernel(x, conv_w, conv_b, sn_u, bn_gamma, bn_beta):
    N, H, W, Cin = x.shape
    Cout = conv_w.shape[-1]
    H2, W2 = 2 * H, 2 * W
    C2 = 2 * Cout

    u_row = sn_u.reshape(1, Cout)
    b_row = conv_b.reshape(1, Cout)
    g_row = bn_gamma.reshape(1, Cout)
    bt_row = bn_beta.reshape(1, Cout)

    vmem_limit = 100 * 1024 * 1024

    stats, w2 = pl.pallas_call(
        functools.partial(_stats_kernel, H=H, W=W, Cin=Cin, C2=C2),
        out_shape=(
            jax.ShapeDtypeStruct((2, C2), jnp.float32),
            jax.ShapeDtypeStruct((6, 2 * Cin, C2), jnp.bfloat16),
        ),
        grid=(N,),
        in_specs=[
            pl.BlockSpec((1, H, W, Cin), lambda n: (n, 0, 0, 0)),
            pl.BlockSpec((3, 3, Cin, Cout), lambda n: (0, 0, 0, 0)),
            pl.BlockSpec((1, Cout), lambda n: (0, 0)),
            pl.BlockSpec((1, Cout), lambda n: (0, 0)),
        ],
        out_specs=(
            pl.BlockSpec((2, C2), lambda n: (0, 0)),
            pl.BlockSpec((6, 2 * Cin, C2), lambda n: (0, 0, 0)),
        ),
        compiler_params=pltpu.CompilerParams(
            dimension_semantics=("arbitrary",),
            vmem_limit_bytes=vmem_limit),
    )(x, conv_w, u_row, b_row)

    out = pl.pallas_call(
        functools.partial(_apply_kernel, H=H, W=W, Cin=Cin, C2=C2,
                          cnt=float(N * H2 * W2)),
        out_shape=jax.ShapeDtypeStruct((N, H, 2, W, C2), jnp.float32),
        grid=(N,),
        in_specs=[
            pl.BlockSpec((1, H, W, Cin), lambda n: (n, 0, 0, 0)),
            pl.BlockSpec((6, 2 * Cin, C2), lambda n: (0, 0, 0)),
            pl.BlockSpec((2, C2), lambda n: (0, 0)),
            pl.BlockSpec((1, Cout), lambda n: (0, 0)),
            pl.BlockSpec((1, Cout), lambda n: (0, 0)),
            pl.BlockSpec((1, Cout), lambda n: (0, 0)),
        ],
        out_specs=pl.BlockSpec((1, H, 2, W, C2), lambda n: (n, 0, 0, 0, 0)),
        compiler_params=pltpu.CompilerParams(
            dimension_semantics=("arbitrary",),
            vmem_limit_bytes=vmem_limit),
    )(x, w2, stats, g_row, bt_row, b_row)

    return out.reshape(N, H2, W2, Cout)


# speedup vs baseline: 1.4303x; 1.0330x over previous
import functools

import jax
import jax.numpy as jnp
from jax.experimental import pallas as pl
from jax.experimental.pallas import tpu as pltpu

_BN_EPS = 1e-5
_SN_EPS = 1e-12


def _fold_weights(w, u_row):
    Cin, Cout = w.shape[2], w.shape[3]
    wf = w.reshape(9 * Cin, Cout)
    v = jnp.sum(wf * u_row, axis=1, keepdims=True)
    v = v / (jnp.sqrt(jnp.sum(v * v)) + _SN_EPS)
    wv = jnp.sum(wf * v, axis=0, keepdims=True)
    u_new = wv / (jnp.sqrt(jnp.sum(wv * wv)) + _SN_EPS)
    sigma = jnp.sum(u_new * wv)
    w = w / sigma
    d0, d1, d2 = w[:, 0], w[:, 1], w[:, 2]
    z = jnp.zeros_like(d0)
    e = (jnp.concatenate([d0, z], axis=2),
         jnp.concatenate([d1 + d2, d0 + d1], axis=2),
         jnp.concatenate([z, d2], axis=2))
    pieces = []
    for pi in range(2):
        for bb in range(3):
            if pi == 0:
                t0, t1 = e[bb][0], e[bb][1] + e[bb][2]
            else:
                t0, t1 = e[bb][0] + e[bb][1], e[bb][2]
            pieces.append(jnp.concatenate([t0, t1], axis=0))
    return jnp.stack(pieces, axis=0).astype(jnp.bfloat16)


def _conv_accs(x_ref, w_ref, *, NB, H, W, Cin):
    HW = H * W
    S = HW + 2 * W
    M = NB * S
    z = jnp.zeros((W, Cin), jnp.float32)
    pieces = []
    for b in range(NB):
        pieces += [z, x_ref[b].reshape(HW, Cin), z]
    xe = jnp.concatenate(pieces, axis=0)
    xc = jnp.concatenate([xe, jnp.zeros((W, Cin), jnp.float32)], axis=0)
    xc = jnp.concatenate([xc[0:M], xc[W:M + W]], axis=1)
    col = jax.lax.broadcasted_iota(jnp.int32, (M, 2 * Cin), 0)
    col = (col & (W - 1)) if (W & (W - 1)) == 0 else (col % W)
    p1 = xc.astype(jnp.bfloat16)
    p0 = jnp.where(col != 0, pltpu.roll(xc, 1, axis=0), 0.0
                   ).astype(jnp.bfloat16)
    p2 = jnp.where(col != W - 1, pltpu.roll(xc, M - 1, axis=0), 0.0
                   ).astype(jnp.bfloat16)
    span = (NB - 1) * S + HW
    accs = [[None, None] for _ in range(NB)]
    for pi in range(2):
        acc = None
        for bb, p in ((0, p0), (1, p1), (2, p2)):
            d = jnp.dot(p[pi * W:pi * W + span], w_ref[pi * 3 + bb],
                        preferred_element_type=jnp.float32)
            acc = d if acc is None else acc + d
        for b in range(NB):
            accs[b][pi] = acc[b * S:b * S + HW]
    return accs


def _stats_kernel(x_ref, cw_ref, u_ref, b_ref, st_ref, w2_ref,
                  *, NB, H, W, Cin, C2):
    n = pl.program_id(0)

    @pl.when(n == 0)
    def _():
        w2_ref[...] = _fold_weights(cw_ref[...], u_ref[...])
        st_ref[...] = jnp.zeros_like(st_ref)

    accs = _conv_accs(x_ref, w2_ref, NB=NB, H=H, W=W, Cin=Cin)
    b2 = jnp.concatenate([b_ref[...], b_ref[...]], axis=1)
    ssum = None
    ssq = None
    for b in range(NB):
        for pi in range(2):
            a = accs[b][pi] + b2
            s1 = jnp.sum(a, axis=0, keepdims=True)
            s2 = jnp.sum(a * a, axis=0, keepdims=True)
            ssum = s1 if ssum is None else ssum + s1
            ssq = s2 if ssq is None else ssq + s2
    st_ref[...] += jnp.concatenate([ssum, ssq], axis=0)


def _apply_kernel(x_ref, w_ref, st_ref, g_ref, bt_ref, b_ref, o_ref,
                  *, NB, H, W, Cin, C2, cnt):
    Cout = C2 // 2
    st = st_ref[...]
    tot = st[:, 0:Cout] + st[:, Cout:C2]
    mean = tot[0:1] * (1.0 / cnt)
    var = jnp.maximum(tot[1:2] * (1.0 / cnt) - mean * mean, 0.0)
    invstd = jax.lax.rsqrt(var + _BN_EPS)
    scale = g_ref[...] * invstd
    shift = bt_ref[...] + (b_ref[...] - mean) * scale
    sc = jnp.concatenate([scale, scale], axis=1)
    sh = jnp.concatenate([shift, shift], axis=1)

    accs = _conv_accs(x_ref, w_ref, NB=NB, H=H, W=W, Cin=Cin)
    for b in range(NB):
        o0 = jnp.maximum(accs[b][0] * sc + sh, 0.0).reshape(H, W, C2)
        o1 = jnp.maximum(accs[b][1] * sc + sh, 0.0).reshape(H, W, C2)
        o_ref[b] = jnp.stack([o0, o1], axis=1)


def kernel(x, conv_w, conv_b, sn_u, bn_gamma, bn_beta):
    N, H, W, Cin = x.shape
    Cout = conv_w.shape[-1]
    H2, W2 = 2 * H, 2 * W
    C2 = 2 * Cout
    NB = 2 if N % 2 == 0 else 1

    u_row = sn_u.reshape(1, Cout)
    b_row = conv_b.reshape(1, Cout)
    g_row = bn_gamma.reshape(1, Cout)
    bt_row = bn_beta.reshape(1, Cout)

    vmem_limit = 100 * 1024 * 1024

    stats, w2 = pl.pallas_call(
        functools.partial(_stats_kernel, NB=NB, H=H, W=W, Cin=Cin, C2=C2),
        out_shape=(
            jax.ShapeDtypeStruct((2, C2), jnp.float32),
            jax.ShapeDtypeStruct((6, 2 * Cin, C2), jnp.bfloat16),
        ),
        grid=(N // NB,),
        in_specs=[
            pl.BlockSpec((NB, H, W, Cin), lambda n: (n, 0, 0, 0)),
            pl.BlockSpec((3, 3, Cin, Cout), lambda n: (0, 0, 0, 0)),
            pl.BlockSpec((1, Cout), lambda n: (0, 0)),
            pl.BlockSpec((1, Cout), lambda n: (0, 0)),
        ],
        out_specs=(
            pl.BlockSpec((2, C2), lambda n: (0, 0)),
            pl.BlockSpec((6, 2 * Cin, C2), lambda n: (0, 0, 0)),
        ),
        compiler_params=pltpu.CompilerParams(
            dimension_semantics=("arbitrary",),
            vmem_limit_bytes=vmem_limit),
    )(x, conv_w, u_row, b_row)

    out = pl.pallas_call(
        functools.partial(_apply_kernel, NB=NB, H=H, W=W, Cin=Cin, C2=C2,
                          cnt=float(N * H2 * W2)),
        out_shape=jax.ShapeDtypeStruct((N, H, 2, W, C2), jnp.float32),
        grid=(N // NB,),
        in_specs=[
            pl.BlockSpec((NB, H, W, Cin), lambda n: (n, 0, 0, 0)),
            pl.BlockSpec((6, 2 * Cin, C2), lambda n: (0, 0, 0)),
            pl.BlockSpec((2, C2), lambda n: (0, 0)),
            pl.BlockSpec((1, Cout), lambda n: (0, 0)),
            pl.BlockSpec((1, Cout), lambda n: (0, 0)),
            pl.BlockSpec((1, Cout), lambda n: (0, 0)),
        ],
        out_specs=pl.BlockSpec((NB, H, 2, W, C2), lambda n: (n, 0, 0, 0, 0)),
        compiler_params=pltpu.CompilerParams(
            dimension_semantics=("arbitrary",),
            vmem_limit_bytes=vmem_limit),
    )(x, w2, stats, g_row, bt_row, b_row)

    return out.reshape(N, H2, W2, Cout)


# speedup vs baseline: 1.4437x; 1.0094x over previous
import functools

import jax
import jax.numpy as jnp
from jax.experimental import pallas as pl
from jax.experimental.pallas import tpu as pltpu

_BN_EPS = 1e-5
_SN_EPS = 1e-12


def _fold_weights(w, u_row):
    Cin, Cout = w.shape[2], w.shape[3]
    wf = w.reshape(9 * Cin, Cout)
    v = jnp.sum(wf * u_row, axis=1, keepdims=True)
    v = v / (jnp.sqrt(jnp.sum(v * v)) + _SN_EPS)
    wv = jnp.sum(wf * v, axis=0, keepdims=True)
    u_new = wv / (jnp.sqrt(jnp.sum(wv * wv)) + _SN_EPS)
    sigma = jnp.sum(u_new * wv)
    w = w / sigma
    d0, d1, d2 = w[:, 0], w[:, 1], w[:, 2]
    z = jnp.zeros_like(d0)
    e = (jnp.concatenate([d0, z], axis=2),
         jnp.concatenate([d1 + d2, d0 + d1], axis=2),
         jnp.concatenate([z, d2], axis=2))
    pieces = []
    for pi in range(2):
        for bb in range(3):
            if pi == 0:
                t0, t1 = e[bb][0], e[bb][1] + e[bb][2]
            else:
                t0, t1 = e[bb][0] + e[bb][1], e[bb][2]
            pieces.append(jnp.concatenate([t0, t1], axis=0))
    return jnp.stack(pieces, axis=0).astype(jnp.bfloat16)


def _conv_accs(x_ref, w_ref, *, NB, H, W, Cin):
    HW = H * W
    S = HW + 2 * W
    M = NB * S
    z = jnp.zeros((W, Cin), jnp.float32)
    pieces = []
    for b in range(NB):
        pieces += [z, x_ref[b].reshape(HW, Cin), z]
    xe = jnp.concatenate(pieces, axis=0)
    xc = jnp.concatenate([xe, jnp.zeros((W, Cin), jnp.float32)], axis=0)
    xc = jnp.concatenate([xc[0:M], xc[W:M + W]], axis=1)
    col = jax.lax.broadcasted_iota(jnp.int32, (M, 2 * Cin), 0)
    col = (col & (W - 1)) if (W & (W - 1)) == 0 else (col % W)
    p1 = xc.astype(jnp.bfloat16)
    p0 = jnp.where(col != 0, pltpu.roll(xc, 1, axis=0), 0.0
                   ).astype(jnp.bfloat16)
    p2 = jnp.where(col != W - 1, pltpu.roll(xc, M - 1, axis=0), 0.0
                   ).astype(jnp.bfloat16)
    span = (NB - 1) * S + HW
    accs = [[None, None] for _ in range(NB)]
    for pi in range(2):
        acc = None
        for bb, p in ((0, p0), (1, p1), (2, p2)):
            d = jnp.dot(p[pi * W:pi * W + span], w_ref[pi * 3 + bb],
                        preferred_element_type=jnp.float32)
            acc = d if acc is None else acc + d
        for b in range(NB):
            accs[b][pi] = acc[b * S:b * S + HW]
    return accs


def _stats_kernel(x_ref, cw_ref, u_ref, st_ref, w2_ref,
                  *, NB, H, W, Cin, C2):
    n = pl.program_id(0)

    @pl.when(n == 0)
    def _():
        w2_ref[...] = _fold_weights(cw_ref[...], u_ref[...])
        st_ref[...] = jnp.zeros_like(st_ref)

    accs = _conv_accs(x_ref, w2_ref, NB=NB, H=H, W=W, Cin=Cin)
    ssum = None
    ssq = None
    for b in range(NB):
        for pi in range(2):
            a = accs[b][pi]
            s1 = jnp.sum(a, axis=0, keepdims=True)
            s2 = jnp.sum(a * a, axis=0, keepdims=True)
            ssum = s1 if ssum is None else ssum + s1
            ssq = s2 if ssq is None else ssq + s2
    st_ref[...] += jnp.concatenate([ssum, ssq], axis=0)


def _apply_kernel(x_ref, w_ref, st_ref, g_ref, bt_ref, b_ref, o_ref,
                  *, NB, H, W, Cin, C2, cnt):
    Cout = C2 // 2
    st = st_ref[...]
    tot = st[:, 0:Cout] + st[:, Cout:C2]
    m0 = tot[0:1] * (1.0 / cnt)
    var = jnp.maximum(tot[1:2] * (1.0 / cnt) - m0 * m0, 0.0)
    invstd = jax.lax.rsqrt(var + _BN_EPS)
    scale = g_ref[...] * invstd
    shift = bt_ref[...] - m0 * scale
    sc = jnp.concatenate([scale, scale], axis=1)
    sh = jnp.concatenate([shift, shift], axis=1)

    accs = _conv_accs(x_ref, w_ref, NB=NB, H=H, W=W, Cin=Cin)
    for b in range(NB):
        o_ref[b, :, 0] = jnp.maximum(accs[b][0] * sc + sh, 0.0
                                     ).reshape(H, W, C2)
        o_ref[b, :, 1] = jnp.maximum(accs[b][1] * sc + sh, 0.0
                                     ).reshape(H, W, C2)


def kernel(x, conv_w, conv_b, sn_u, bn_gamma, bn_beta):
    N, H, W, Cin = x.shape
    Cout = conv_w.shape[-1]
    H2, W2 = 2 * H, 2 * W
    C2 = 2 * Cout
    NB = 2 if N % 2 == 0 else 1

    u_row = sn_u.reshape(1, Cout)
    b_row = conv_b.reshape(1, Cout)
    g_row = bn_gamma.reshape(1, Cout)
    bt_row = bn_beta.reshape(1, Cout)

    vmem_limit = 100 * 1024 * 1024

    stats, w2 = pl.pallas_call(
        functools.partial(_stats_kernel, NB=NB, H=H, W=W, Cin=Cin, C2=C2),
        out_shape=(
            jax.ShapeDtypeStruct((2, C2), jnp.float32),
            jax.ShapeDtypeStruct((6, 2 * Cin, C2), jnp.bfloat16),
        ),
        grid=(N // NB,),
        in_specs=[
            pl.BlockSpec((NB, H, W, Cin), lambda n: (n, 0, 0, 0)),
            pl.BlockSpec((3, 3, Cin, Cout), lambda n: (0, 0, 0, 0)),
            pl.BlockSpec((1, Cout), lambda n: (0, 0)),
        ],
        out_specs=(
            pl.BlockSpec((2, C2), lambda n: (0, 0)),
            pl.BlockSpec((6, 2 * Cin, C2), lambda n: (0, 0, 0)),
        ),
        compiler_params=pltpu.CompilerParams(
            dimension_semantics=("arbitrary",),
            vmem_limit_bytes=vmem_limit),
    )(x, conv_w, u_row)

    out = pl.pallas_call(
        functools.partial(_apply_kernel, NB=NB, H=H, W=W, Cin=Cin, C2=C2,
                          cnt=float(N * H2 * W2)),
        out_shape=jax.ShapeDtypeStruct((N, H, 2, W, C2), jnp.float32),
        grid=(N // NB,),
        in_specs=[
            pl.BlockSpec((NB, H, W, Cin), lambda n: (n, 0, 0, 0)),
            pl.BlockSpec((6, 2 * Cin, C2), lambda n: (0, 0, 0)),
            pl.BlockSpec((2, C2), lambda n: (0, 0)),
            pl.BlockSpec((1, Cout), lambda n: (0, 0)),
            pl.BlockSpec((1, Cout), lambda n: (0, 0)),
            pl.BlockSpec((1, Cout), lambda n: (0, 0)),
        ],
        out_specs=pl.BlockSpec((NB, H, 2, W, C2), lambda n: (n, 0, 0, 0, 0)),
        compiler_params=pltpu.CompilerParams(
            dimension_semantics=("arbitrary",),
            vmem_limit_bytes=vmem_limit),
    )(x, w2, stats, g_row, bt_row, b_row)

    return out.reshape(N, H2, W2, Cout)


# speedup vs baseline: 1.4461x; 1.0017x over previous
import functools

import jax
import jax.numpy as jnp
from jax.experimental import pallas as pl
from jax.experimental.pallas import tpu as pltpu

_BN_EPS = 1e-5
_SN_EPS = 1e-12


def _fold_weights(w, u_row):
    Cin, Cout = w.shape[2], w.shape[3]
    wf = w.reshape(9 * Cin, Cout)
    v = jnp.sum(wf * u_row, axis=1, keepdims=True)
    v = v / (jnp.sqrt(jnp.sum(v * v)) + _SN_EPS)
    wv = jnp.sum(wf * v, axis=0, keepdims=True)
    u_new = wv / (jnp.sqrt(jnp.sum(wv * wv)) + _SN_EPS)
    sigma = jnp.sum(u_new * wv)
    w = w / sigma
    d0, d1, d2 = w[:, 0], w[:, 1], w[:, 2]
    z = jnp.zeros_like(d0)
    e = (jnp.concatenate([d0, z], axis=2),
         jnp.concatenate([d1 + d2, d0 + d1], axis=2),
         jnp.concatenate([z, d2], axis=2))
    pieces = []
    for pi in range(2):
        for bb in range(3):
            if pi == 0:
                t0, t1 = e[bb][0], e[bb][1] + e[bb][2]
            else:
                t0, t1 = e[bb][0] + e[bb][1], e[bb][2]
            pieces.append(jnp.concatenate([t0, t1], axis=0))
    return jnp.stack(pieces, axis=0).astype(jnp.bfloat16)


def _conv_accs(x_ref, w_ref, *, NB, H, W, Cin):
    HW = H * W
    S = HW + 2 * W
    M = NB * S
    z = jnp.zeros((W, Cin), jnp.float32)
    pieces = []
    for b in range(NB):
        pieces += [z, x_ref[b].reshape(HW, Cin), z]
    xe = jnp.concatenate(pieces, axis=0)
    xc = jnp.concatenate([xe, jnp.zeros((W, Cin), jnp.float32)], axis=0)
    xc = jnp.concatenate([xc[0:M], xc[W:M + W]], axis=1)
    col = jax.lax.broadcasted_iota(jnp.int32, (M, 2 * Cin), 0)
    col = (col & (W - 1)) if (W & (W - 1)) == 0 else (col % W)
    p1 = xc.astype(jnp.bfloat16)
    p0 = jnp.where(col != 0, pltpu.roll(xc, 1, axis=0), 0.0
                   ).astype(jnp.bfloat16)
    p2 = jnp.where(col != W - 1, pltpu.roll(xc, M - 1, axis=0), 0.0
                   ).astype(jnp.bfloat16)
    span = (NB - 1) * S + HW
    accs = [[None, None] for _ in range(NB)]
    for pi in range(2):
        acc = None
        for bb, p in ((0, p0), (1, p1), (2, p2)):
            d = jnp.dot(p[pi * W:pi * W + span], w_ref[pi * 3 + bb],
                        preferred_element_type=jnp.float32)
            acc = d if acc is None else acc + d
        for b in range(NB):
            accs[b][pi] = acc[b * S:b * S + HW]
    return accs


def _stats_kernel(x_ref, cw_ref, u_ref, st_ref, w2_ref,
                  *, NB, H, W, Cin, C2):
    n = pl.program_id(0)

    @pl.when(n == 0)
    def _():
        w2_ref[...] = _fold_weights(cw_ref[...], u_ref[...])
        st_ref[...] = jnp.zeros_like(st_ref)

    accs = _conv_accs(x_ref, w2_ref, NB=NB, H=H, W=W, Cin=Cin)
    ssum = None
    ssq = None
    for b in range(NB):
        for pi in range(2):
            a = accs[b][pi]
            s1 = jnp.sum(a, axis=0, keepdims=True)
            s2 = jnp.sum(a * a, axis=0, keepdims=True)
            ssum = s1 if ssum is None else ssum + s1
            ssq = s2 if ssq is None else ssq + s2
    st_ref[...] += jnp.concatenate([ssum, ssq], axis=0)


def _apply_kernel(x_ref, w_ref, st_ref, g_ref, bt_ref, o_ref,
                  *, NB, H, W, Cin, C2, cnt):
    Cout = C2 // 2
    st = st_ref[...]
    tot = st[:, 0:Cout] + st[:, Cout:C2]
    m0 = tot[0:1] * (1.0 / cnt)
    var = jnp.maximum(tot[1:2] * (1.0 / cnt) - m0 * m0, 0.0)
    invstd = jax.lax.rsqrt(var + _BN_EPS)
    scale = g_ref[...] * invstd
    shift = bt_ref[...] - m0 * scale
    sc = jnp.concatenate([scale, scale], axis=1)
    sh = jnp.concatenate([shift, shift], axis=1)

    accs = _conv_accs(x_ref, w_ref, NB=NB, H=H, W=W, Cin=Cin)
    for b in range(NB):
        o_ref[b, :, 0] = jnp.maximum(accs[b][0] * sc + sh, 0.0
                                     ).reshape(H, W, C2)
        o_ref[b, :, 1] = jnp.maximum(accs[b][1] * sc + sh, 0.0
                                     ).reshape(H, W, C2)


def kernel(x, conv_w, conv_b, sn_u, bn_gamma, bn_beta):
    N, H, W, Cin = x.shape
    Cout = conv_w.shape[-1]
    H2, W2 = 2 * H, 2 * W
    C2 = 2 * Cout
    NB1 = 4 if N % 4 == 0 else (2 if N % 2 == 0 else 1)
    NB2 = 2 if N % 2 == 0 else 1

    u_row = sn_u.reshape(1, Cout)
    g_row = bn_gamma.reshape(1, Cout)
    bt_row = bn_beta.reshape(1, Cout)

    vmem_limit = 100 * 1024 * 1024

    stats, w2 = pl.pallas_call(
        functools.partial(_stats_kernel, NB=NB1, H=H, W=W, Cin=Cin, C2=C2),
        out_shape=(
            jax.ShapeDtypeStruct((2, C2), jnp.float32),
            jax.ShapeDtypeStruct((6, 2 * Cin, C2), jnp.bfloat16),
        ),
        grid=(N // NB1,),
        in_specs=[
            pl.BlockSpec((NB1, H, W, Cin), lambda n: (n, 0, 0, 0)),
            pl.BlockSpec((3, 3, Cin, Cout), lambda n: (0, 0, 0, 0)),
            pl.BlockSpec((1, Cout), lambda n: (0, 0)),
        ],
        out_specs=(
            pl.BlockSpec((2, C2), lambda n: (0, 0)),
            pl.BlockSpec((6, 2 * Cin, C2), lambda n: (0, 0, 0)),
        ),
        compiler_params=pltpu.CompilerParams(
            dimension_semantics=("arbitrary",),
            vmem_limit_bytes=vmem_limit),
    )(x, conv_w, u_row)

    out = pl.pallas_call(
        functools.partial(_apply_kernel, NB=NB2, H=H, W=W, Cin=Cin, C2=C2,
                          cnt=float(N * H2 * W2)),
        out_shape=jax.ShapeDtypeStruct((N, H, 2, W, C2), jnp.float32),
        grid=(N // NB2,),
        in_specs=[
            pl.BlockSpec((NB2, H, W, Cin), lambda n: (n, 0, 0, 0)),
            pl.BlockSpec((6, 2 * Cin, C2), lambda n: (0, 0, 0)),
            pl.BlockSpec((2, C2), lambda n: (0, 0)),
            pl.BlockSpec((1, Cout), lambda n: (0, 0)),
            pl.BlockSpec((1, Cout), lambda n: (0, 0)),
        ],
        out_specs=pl.BlockSpec((NB2, H, 2, W, C2), lambda n: (n, 0, 0, 0, 0)),
        compiler_params=pltpu.CompilerParams(
            dimension_semantics=("arbitrary",),
            vmem_limit_bytes=vmem_limit),
    )(x, w2, stats, g_row, bt_row)

    return out.reshape(N, H2, W2, Cout)


# speedup vs baseline: 1.8382x; 1.2711x over previous
import functools

import jax
import jax.numpy as jnp
from jax.experimental import pallas as pl
from jax.experimental.pallas import tpu as pltpu

_BN_EPS = 1e-5
_SN_EPS = 1e-12


def _fold_weights(w, u_row):
    Cin, Cout = w.shape[2], w.shape[3]
    wf = w.reshape(9 * Cin, Cout)
    v = jnp.sum(wf * u_row, axis=1, keepdims=True)
    v = v / (jnp.sqrt(jnp.sum(v * v)) + _SN_EPS)
    wv = jnp.sum(wf * v, axis=0, keepdims=True)
    u_new = wv / (jnp.sqrt(jnp.sum(wv * wv)) + _SN_EPS)
    sigma = jnp.sum(u_new * wv)
    w = w / sigma
    d0, d1, d2 = w[:, 0], w[:, 1], w[:, 2]
    z = jnp.zeros_like(d0)
    e = (jnp.concatenate([d0, z], axis=2),
         jnp.concatenate([d1 + d2, d0 + d1], axis=2),
         jnp.concatenate([z, d2], axis=2))
    pieces = []
    for pi in range(2):
        for bb in range(3):
            if pi == 0:
                t0, t1 = e[bb][0], e[bb][1] + e[bb][2]
            else:
                t0, t1 = e[bb][0] + e[bb][1], e[bb][2]
            pieces.append(jnp.concatenate([t0, t1], axis=0))
    return jnp.stack(pieces, axis=0).astype(jnp.bfloat16)


def _conv_accs(x_ref, w_ref, *, NB, H, W, Cin):
    HW = H * W
    S = HW + 2 * W
    M = NB * S
    z = jnp.zeros((W, Cin), jnp.float32)
    pieces = []
    for b in range(NB):
        pieces += [z, x_ref[b].reshape(HW, Cin), z]
    xe = jnp.concatenate(pieces, axis=0)
    xc = jnp.concatenate([xe, jnp.zeros((W, Cin), jnp.float32)], axis=0)
    xc = jnp.concatenate([xc[0:M], xc[W:M + W]], axis=1)
    col = jax.lax.broadcasted_iota(jnp.int32, (M, 2 * Cin), 0)
    col = (col & (W - 1)) if (W & (W - 1)) == 0 else (col % W)
    p1 = xc.astype(jnp.bfloat16)
    p0 = jnp.where(col != 0, pltpu.roll(xc, 1, axis=0), 0.0
                   ).astype(jnp.bfloat16)
    p2 = jnp.where(col != W - 1, pltpu.roll(xc, M - 1, axis=0), 0.0
                   ).astype(jnp.bfloat16)
    span = (NB - 1) * S + HW
    accs = [[None, None] for _ in range(NB)]
    for pi in range(2):
        acc = None
        for bb, p in ((0, p0), (1, p1), (2, p2)):
            d = jnp.dot(p[pi * W:pi * W + span], w_ref[pi * 3 + bb],
                        preferred_element_type=jnp.float32)
            acc = d if acc is None else acc + d
        for b in range(NB):
            accs[b][pi] = acc[b * S:b * S + HW]
    return accs


def _stats_kernel(x_ref, cw_ref, u_ref, st_ref, w2_ref,
                  *, NB, H, W, Cin, C2):
    n = pl.program_id(0)

    @pl.when(n == 0)
    def _():
        w2_ref[...] = _fold_weights(cw_ref[...], u_ref[...])
        st_ref[...] = jnp.zeros_like(st_ref)

    accs = _conv_accs(x_ref, w2_ref, NB=NB, H=H, W=W, Cin=Cin)
    ssum = None
    ssq = None
    for b in range(NB):
        for pi in range(2):
            a = accs[b][pi]
            s1 = jnp.sum(a, axis=0, keepdims=True)
            s2 = jnp.sum(a * a, axis=0, keepdims=True)
            ssum = s1 if ssum is None else ssum + s1
            ssq = s2 if ssq is None else ssq + s2
    st_ref[...] += jnp.concatenate([ssum, ssq], axis=0)


def _apply_kernel(x_ref, w_ref, st_ref, g_ref, bt_ref, o_ref,
                  *, NB, H, W, Cin, C2, cnt):
    Cout = C2 // 2
    st = st_ref[...]
    tot = st[:, 0:Cout] + st[:, Cout:C2]
    m0 = tot[0:1] * (1.0 / cnt)
    var = jnp.maximum(tot[1:2] * (1.0 / cnt) - m0 * m0, 0.0)
    invstd = jax.lax.rsqrt(var + _BN_EPS)
    scale = g_ref[...] * invstd
    shift = bt_ref[...] - m0 * scale
    sc = jnp.concatenate([scale, scale], axis=1)
    sh = jnp.concatenate([shift, shift], axis=1)

    accs = _conv_accs(x_ref, w_ref, NB=NB, H=H, W=W, Cin=Cin)
    for b in range(NB):
        for pi in range(2):
            o = jnp.maximum(accs[b][pi] * sc + sh, 0.0)
            for pj in range(2):
                opj = o[:, pj * Cout:(pj + 1) * Cout].reshape(H, W, Cout)
                o_ref[b, pl.ds(pi, H, 2), pl.ds(pj, W, 2)] = opj


def kernel(x, conv_w, conv_b, sn_u, bn_gamma, bn_beta):
    N, H, W, Cin = x.shape
    Cout = conv_w.shape[-1]
    H2, W2 = 2 * H, 2 * W
    C2 = 2 * Cout
    NB1 = 4 if N % 4 == 0 else (2 if N % 2 == 0 else 1)
    NB2 = 2 if N % 2 == 0 else 1

    u_row = sn_u.reshape(1, Cout)
    g_row = bn_gamma.reshape(1, Cout)
    bt_row = bn_beta.reshape(1, Cout)

    vmem_limit = 100 * 1024 * 1024

    stats, w2 = pl.pallas_call(
        functools.partial(_stats_kernel, NB=NB1, H=H, W=W, Cin=Cin, C2=C2),
        out_shape=(
            jax.ShapeDtypeStruct((2, C2), jnp.float32),
            jax.ShapeDtypeStruct((6, 2 * Cin, C2), jnp.bfloat16),
        ),
        grid=(N // NB1,),
        in_specs=[
            pl.BlockSpec((NB1, H, W, Cin), lambda n: (n, 0, 0, 0)),
            pl.BlockSpec((3, 3, Cin, Cout), lambda n: (0, 0, 0, 0)),
            pl.BlockSpec((1, Cout), lambda n: (0, 0)),
        ],
        out_specs=(
            pl.BlockSpec((2, C2), lambda n: (0, 0)),
            pl.BlockSpec((6, 2 * Cin, C2), lambda n: (0, 0, 0)),
        ),
        compiler_params=pltpu.CompilerParams(
            dimension_semantics=("arbitrary",),
            vmem_limit_bytes=vmem_limit),
    )(x, conv_w, u_row)

    out = pl.pallas_call(
        functools.partial(_apply_kernel, NB=NB2, H=H, W=W, Cin=Cin, C2=C2,
                          cnt=float(N * H2 * W2)),
        out_shape=jax.ShapeDtypeStruct((N, H2, W2, Cout), jnp.float32),
        grid=(N // NB2,),
        in_specs=[
            pl.BlockSpec((NB2, H, W, Cin), lambda n: (n, 0, 0, 0)),
            pl.BlockSpec((6, 2 * Cin, C2), lambda n: (0, 0, 0)),
            pl.BlockSpec((2, C2), lambda n: (0, 0)),
            pl.BlockSpec((1, Cout), lambda n: (0, 0)),
            pl.BlockSpec((1, Cout), lambda n: (0, 0)),
        ],
        out_specs=pl.BlockSpec((NB2, H2, W2, Cout), lambda n: (n, 0, 0, 0)),
        compiler_params=pltpu.CompilerParams(
            dimension_semantics=("arbitrary",),
            vmem_limit_bytes=vmem_limit),
    )(x, w2, stats, g_row, bt_row)

    return out
